# baseline (speedup 1.0000x reference)
import jax
import jax.numpy as jnp
from jax import lax
from jax.experimental import pallas as pl
from jax.experimental.pallas import tpu as pltpu
from jax.experimental.pallas import tpu_sc as plsc

B = 16384
SPARSE_VOCAB = 1000
VOCAB_PAD = 1024
EMB_DIM = 64
HID = 50
OUT = 50
OUT_PAD = 64

DENSE_PAIRS = 500000

NC = 2
NS = 16
NW = NC * NS
B_PER_W = B // NW
CHUNK = 128
N_CHUNK = B_PER_W // CHUNK


def _precompute_body(ws1e_ref, ws1o_ref, bs1_ref, ws2_ref, bs2_ref, t_ref):
    he = jnp.maximum(ws1e_ref[...] + bs1_ref[...], 0.0)
    ho = jnp.maximum(ws1o_ref[...] + bs1_ref[...], 0.0)
    t_ref[:, :OUT_PAD] = (
        jnp.dot(he, ws2_ref[...], preferred_element_type=jnp.float32)
        + bs2_ref[...]
    )
    t_ref[:, OUT_PAD:] = (
        jnp.dot(ho, ws2_ref[...], preferred_element_type=jnp.float32)
        + bs2_ref[...]
    )


def _precompute_table(ws1e, ws1o, bs1, ws2p, bs2p):
    return pl.pallas_call(
        _precompute_body,
        out_shape=jax.ShapeDtypeStruct((VOCAB_PAD // 2, 2 * OUT_PAD),
                                       jnp.float32),
    )(ws1e, ws1o, bs1, ws2p, bs2p)


def _sc_gather_body(emb_hbm, didx_hbm, t_hbm, sidx_hbm,
                    demb_out, srow_out,
                    didx_v, sidx_v, rows_v, sem):
    wid = lax.axis_index("s") * NC + lax.axis_index("c")
    base = wid * B_PER_W
    row0 = wid * N_CHUNK
    pltpu.sync_copy(didx_hbm.at[pl.ds(row0, N_CHUNK)], didx_v)
    pltpu.sync_copy(sidx_hbm.at[pl.ds(row0, N_CHUNK)], sidx_v)
    copies = []
    for j in range(N_CHUNK):
        copies.append(pltpu.async_copy(
            emb_hbm.at[didx_v.at[j]],
            rows_v.at[pl.ds(j * CHUNK, CHUNK)], sem))
    for cp in copies:
        cp.wait()
    pltpu.sync_copy(rows_v, demb_out.at[pl.ds(base, B_PER_W)])
    copies = []
    for j in range(N_CHUNK):
        copies.append(pltpu.async_copy(
            t_hbm.at[sidx_v.at[j]],
            rows_v.at[pl.ds(j * CHUNK, CHUNK)], sem))
    for cp in copies:
        cp.wait()
    pltpu.sync_copy(rows_v, srow_out.at[pl.ds(base, B_PER_W)])


def _sc_gather(emb_pairs, didx2d, t_table, sidx2d):
    mesh = plsc.VectorSubcoreMesh(core_axis_name="c", subcore_axis_name="s")
    return pl.kernel(
        _sc_gather_body,
        mesh=mesh,
        out_type=[
            jax.ShapeDtypeStruct((B, 2 * EMB_DIM), jnp.float32),
            jax.ShapeDtypeStruct((B, 2 * OUT_PAD), jnp.float32),
        ],
        scratch_types=[
            pltpu.VMEM((N_CHUNK, CHUNK), jnp.int32),
            pltpu.VMEM((N_CHUNK, CHUNK), jnp.int32),
            pltpu.VMEM((B_PER_W, 2 * EMB_DIM), jnp.float32),
            pltpu.SemaphoreType.DMA,
        ],
    )(emb_pairs, didx2d, t_table, sidx2d)


_MLP_BLOCK = 2048


def _mlp_body(sg_ref, dg_ref, spar_ref, dpar_ref,
              wd1_ref, bd1_ref, wd2_ref, bd2_ref,
              sout_ref, dout_ref):
    spar = spar_ref[...]
    sout_ref[...] = (sg_ref[:, :OUT] * (1.0 - spar)
                     + sg_ref[:, OUT_PAD:OUT_PAD + OUT] * spar)
    dpar = dpar_ref[...]
    x = dg_ref[:, :EMB_DIM] * (1.0 - dpar) + dg_ref[:, EMB_DIM:] * dpar
    h = jnp.maximum(
        jnp.dot(x, wd1_ref[...], preferred_element_type=jnp.float32)
        + bd1_ref[...], 0.0)
    dout_ref[...] = (
        jnp.dot(h, wd2_ref[...], preferred_element_type=jnp.float32)
        + bd2_ref[...]
    )


def _mlp(s_rows, d_rows, spar, dpar, wd1, bd1, wd2, bd2):
    nblk = B // _MLP_BLOCK
    return pl.pallas_call(
        _mlp_body,
        grid=(nblk,),
        in_specs=[
            pl.BlockSpec((_MLP_BLOCK, 2 * OUT_PAD), lambda i: (i, 0)),
            pl.BlockSpec((_MLP_BLOCK, 2 * EMB_DIM), lambda i: (i, 0)),
            pl.BlockSpec((_MLP_BLOCK, 1), lambda i: (i, 0)),
            pl.BlockSpec((_MLP_BLOCK, 1), lambda i: (i, 0)),
            pl.BlockSpec((EMB_DIM, HID), lambda i: (0, 0)),
            pl.BlockSpec((1, HID), lambda i: (0, 0)),
            pl.BlockSpec((HID, OUT), lambda i: (0, 0)),
            pl.BlockSpec((1, OUT), lambda i: (0, 0)),
        ],
        out_specs=[
            pl.BlockSpec((_MLP_BLOCK, OUT), lambda i: (i, 0)),
            pl.BlockSpec((_MLP_BLOCK, OUT), lambda i: (i, 0)),
        ],
        out_shape=[
            jax.ShapeDtypeStruct((B, OUT), jnp.float32),
            jax.ShapeDtypeStruct((B, OUT), jnp.float32),
        ],
    )(s_rows, d_rows, spar, dpar, wd1, bd1, wd2, bd2)


def kernel(sparse_col_inp, dense_col_inp, emb_table, Ws1, bs1, Ws2, bs2,
           Wd1, bd1, Wd2, bd2):
    sidx = sparse_col_inp.astype(jnp.int32)
    didx = dense_col_inp.astype(jnp.int32)
    sidx2 = (sidx >> 1).reshape(B // CHUNK, CHUNK)
    didx2 = (didx >> 1).reshape(B // CHUNK, CHUNK)
    spar = (sidx & 1).astype(jnp.float32).reshape(B, 1)
    dpar = (didx & 1).astype(jnp.float32).reshape(B, 1)
    emb_pairs = emb_table.reshape(DENSE_PAIRS, 2 * EMB_DIM)
    ws1p = jnp.pad(Ws1, ((0, VOCAB_PAD - SPARSE_VOCAB), (0, 0)))
    ws2p = jnp.pad(Ws2, ((0, 0), (0, OUT_PAD - OUT)))
    bs2p = jnp.pad(bs2, (0, OUT_PAD - OUT))
    t_table = _precompute_table(ws1p[0::2], ws1p[1::2], bs1.reshape(1, HID),
                                ws2p, bs2p.reshape(1, OUT_PAD))
    d_rows, s_rows = _sc_gather(emb_pairs, didx2, t_table, sidx2)
    sparse_out, dense_out = _mlp(s_rows, d_rows, spar, dpar,
                                 Wd1, bd1.reshape(1, HID),
                                 Wd2, bd2.reshape(1, OUT))
    return (sparse_out, dense_out)


# speedup vs baseline: 1.5314x; 1.5314x over previous
import jax
import jax.numpy as jnp
from jax import lax
from jax.experimental import pallas as pl
from jax.experimental.pallas import tpu as pltpu
from jax.experimental.pallas import tpu_sc as plsc

B = 16384
SPARSE_VOCAB = 1000
VOCAB_PAD = 1024
DENSE_VOCAB = 1000000
EMB_DIM = 64
HID = 50
OUT = 50
T_ROW = 128
GRP = 8
LANES = 16

NC = 2
NS = 16
NW = NC * NS
B_PER_W = B // NW
CHUNK = 128
N_CHUNK = B_PER_W // CHUNK
GCHUNK = 32
N_ROUND = B_PER_W // GCHUNK


def _precompute_body(ws1_ref, bs1_ref, ws2_ref, bs2_ref, t_ref):
    h = jnp.maximum(ws1_ref[...] + bs1_ref[...], 0.0)
    t_ref[...] = (
        jnp.dot(h, ws2_ref[...], preferred_element_type=jnp.float32)
        + bs2_ref[...]
    )


def _precompute_table(ws1p, bs1, ws2p, bs2p):
    return pl.pallas_call(
        _precompute_body,
        out_shape=jax.ShapeDtypeStruct((VOCAB_PAD, T_ROW), jnp.float32),
    )(ws1p, bs1, ws2p, bs2p)


def _sc_gather_body(emb_hbm, didx_hbm, t_hbm, sidx_hbm,
                    demb_out, srow_out,
                    didx_v, sidx_v, dgrp_v, rows_v,
                    dsem, ssem):
    wid = lax.axis_index("s") * NC + lax.axis_index("c")
    base = wid * B_PER_W
    row0 = wid * N_CHUNK
    pltpu.sync_copy(didx_hbm.at[pl.ds(base, B_PER_W)], didx_v)
    pltpu.sync_copy(sidx_hbm.at[pl.ds(row0, N_CHUNK)], sidx_v)

    scopies = [pltpu.async_copy(
        t_hbm.at[sidx_v.at[j]],
        rows_v.at[pl.ds(j * CHUNK, CHUNK)], ssem) for j in range(N_CHUNK)]

    emb_view = emb_hbm.reshape(DENSE_VOCAB // GRP, GRP, EMB_DIM)

    def fire_round(j0):
        for b in range(GCHUNK // LANES):
            vec = didx_v[pl.ds(j0 + b * LANES, LANES)]
            for i in range(LANES):
                g = vec[i] // GRP
                pltpu.async_copy(emb_view.at[pl.ds(g, 1)],
                                 dgrp_v.at[pl.ds(b * LANES + i, 1)], dsem)

    def drain_round():
        pltpu.make_async_copy(emb_view.at[pl.ds(0, GCHUNK)],
                              dgrp_v, dsem).wait()

    def select_round(j0):
        for b in range(GCHUNK // LANES):
            vec = didx_v[pl.ds(j0 + b * LANES, LANES)]
            for i in range(LANES):
                sub = lax.rem(vec[i], GRP)
                for q in range(EMB_DIM // LANES):
                    rows_v[j0 + b * LANES + i, pl.ds(q * LANES, LANES)] = (
                        dgrp_v[b * LANES + i, sub, pl.ds(q * LANES, LANES)])

    fire_round(0)
    for cp in scopies:
        cp.wait()
    pltpu.sync_copy(rows_v, srow_out.at[pl.ds(base, B_PER_W)])

    def round_body(r, carry):
        j0 = r * GCHUNK
        drain_round()
        select_round(j0)

        @pl.when(r + 1 < N_ROUND)
        def _():
            fire_round(j0 + GCHUNK)

        return carry

    lax.fori_loop(0, N_ROUND, round_body, 0)
    pltpu.sync_copy(rows_v, demb_out.at[pl.ds(base, B_PER_W)])


def _sc_gather(emb_table, didx, t_table, sidx2d):
    mesh = plsc.VectorSubcoreMesh(core_axis_name="c", subcore_axis_name="s")
    return pl.kernel(
        _sc_gather_body,
        mesh=mesh,
        out_type=[
            jax.ShapeDtypeStruct((B, T_ROW), jnp.float32),
            jax.ShapeDtypeStruct((B, T_ROW), jnp.float32),
        ],
        scratch_types=[
            pltpu.VMEM((B_PER_W,), jnp.int32),
            pltpu.VMEM((N_CHUNK, CHUNK), jnp.int32),
            pltpu.VMEM((GCHUNK, GRP, EMB_DIM), jnp.float32),
            pltpu.VMEM((B_PER_W, T_ROW), jnp.float32),
            pltpu.SemaphoreType.DMA,
            pltpu.SemaphoreType.DMA,
        ],
    )(emb_table, didx, t_table, sidx2d)


_MLP_BLOCK = 2048


def _mlp_body(sg_ref, dg_ref, wd1_ref, bd1_ref, wd2_ref, bd2_ref,
              sout_ref, dout_ref):
    sout_ref[...] = sg_ref[:, :OUT]
    h = jnp.maximum(
        jnp.dot(dg_ref[:, :EMB_DIM], wd1_ref[...],
                preferred_element_type=jnp.float32)
        + bd1_ref[...], 0.0)
    dout_ref[...] = (
        jnp.dot(h, wd2_ref[...], preferred_element_type=jnp.float32)
        + bd2_ref[...]
    )


def _mlp(s_rows, d_rows, wd1, bd1, wd2, bd2):
    nblk = B // _MLP_BLOCK
    return pl.pallas_call(
        _mlp_body,
        grid=(nblk,),
        in_specs=[
            pl.BlockSpec((_MLP_BLOCK, T_ROW), lambda i: (i, 0)),
            pl.BlockSpec((_MLP_BLOCK, T_ROW), lambda i: (i, 0)),
            pl.BlockSpec((EMB_DIM, HID), lambda i: (0, 0)),
            pl.BlockSpec((1, HID), lambda i: (0, 0)),
            pl.BlockSpec((HID, OUT), lambda i: (0, 0)),
            pl.BlockSpec((1, OUT), lambda i: (0, 0)),
        ],
        out_specs=[
            pl.BlockSpec((_MLP_BLOCK, OUT), lambda i: (i, 0)),
            pl.BlockSpec((_MLP_BLOCK, OUT), lambda i: (i, 0)),
        ],
        out_shape=[
            jax.ShapeDtypeStruct((B, OUT), jnp.float32),
            jax.ShapeDtypeStruct((B, OUT), jnp.float32),
        ],
    )(s_rows, d_rows, wd1, bd1, wd2, bd2)


def kernel(sparse_col_inp, dense_col_inp, emb_table, Ws1, bs1, Ws2, bs2,
           Wd1, bd1, Wd2, bd2):
    sidx = sparse_col_inp.astype(jnp.int32).reshape(B // CHUNK, CHUNK)
    didx = dense_col_inp.astype(jnp.int32)
    ws1p = jnp.pad(Ws1, ((0, VOCAB_PAD - SPARSE_VOCAB), (0, 0)))
    ws2p = jnp.pad(Ws2, ((0, 0), (0, T_ROW - OUT)))
    bs2p = jnp.pad(bs2, (0, T_ROW - OUT))
    t_table = _precompute_table(ws1p, bs1.reshape(1, HID),
                                ws2p, bs2p.reshape(1, T_ROW))
    d_rows, s_rows = _sc_gather(emb_table, didx, t_table, sidx)
    sparse_out, dense_out = _mlp(s_rows, d_rows, Wd1, bd1.reshape(1, HID),
                                 Wd2, bd2.reshape(1, OUT))
    return (sparse_out, dense_out)


# speedup vs baseline: 1.5323x; 1.0006x over previous
import jax
import jax.numpy as jnp
from jax import lax
from jax.experimental import pallas as pl
from jax.experimental.pallas import tpu as pltpu
from jax.experimental.pallas import tpu_sc as plsc

B = 16384
SPARSE_VOCAB = 1000
VOCAB_PAD = 1024
DENSE_VOCAB = 1000000
EMB_DIM = 64
HID = 50
OUT = 50
T_ROW = 128
HID_PAD = 64
QUAD = 4
GQ_ROWS = DENSE_VOCAB // QUAD

NC = 2
NS = 16
NW = NC * NS
B_PER_W = B // NW
CHUNK = 128
N_CHUNK = B_PER_W // CHUNK


def _precompute_body(ws1_ref, bs1_ref, ws2_ref, bs2_ref, t_ref):
    h = jnp.maximum(ws1_ref[...] + bs1_ref[...], 0.0)
    t_ref[...] = (
        jnp.dot(h, ws2_ref[...], preferred_element_type=jnp.float32)
        + bs2_ref[...]
    )


def _precompute_table(ws1p, bs1, ws2p, bs2p):
    return pl.pallas_call(
        _precompute_body,
        out_shape=jax.ShapeDtypeStruct((VOCAB_PAD, T_ROW), jnp.float32),
    )(ws1p, bs1, ws2p, bs2p)


_GBLK = 4096


def _gbuild_body(et_ref, wd1_ref, gq_ref):
    x = lax.dot_general(et_ref[...], wd1_ref[...], (((0,), (0,)), ((), ())),
                        preferred_element_type=jnp.float32)
    xb = lax.bitcast_convert_type(x, jnp.uint32)
    x4 = xb.reshape(_GBLK // QUAD, QUAD, HID_PAD)
    p01 = ((x4[:, 0, :] >> 16) << 16) | (x4[:, 1, :] >> 16)
    p23 = ((x4[:, 2, :] >> 16) << 16) | (x4[:, 3, :] >> 16)
    gq_ref[...] = lax.bitcast_convert_type(
        jnp.concatenate([p01, p23], axis=1), jnp.int32)


def _gbuild(emb_t, wd1p):
    nblk = (DENSE_VOCAB + _GBLK - 1) // _GBLK
    return pl.pallas_call(
        _gbuild_body,
        grid=(nblk,),
        in_specs=[
            pl.BlockSpec((EMB_DIM, _GBLK), lambda i: (0, i)),
            pl.BlockSpec((EMB_DIM, HID_PAD), lambda i: (0, 0)),
        ],
        out_specs=pl.BlockSpec((_GBLK // QUAD, 2 * HID_PAD), lambda i: (i, 0)),
        out_shape=jax.ShapeDtypeStruct((GQ_ROWS, 2 * HID_PAD), jnp.int32),
    )(emb_t, wd1p)


def _sc_gather_body(gq_hbm, qidx_hbm, t_hbm, sidx_hbm,
                    dpack_out, srow_out,
                    qidx_v, sidx_v, rows_v, sem):
    wid = lax.axis_index("s") * NC + lax.axis_index("c")
    base = wid * B_PER_W
    row0 = wid * N_CHUNK
    pltpu.sync_copy(qidx_hbm.at[pl.ds(row0, N_CHUNK)], qidx_v)
    pltpu.sync_copy(sidx_hbm.at[pl.ds(row0, N_CHUNK)], sidx_v)
    copies = [pltpu.async_copy(
        t_hbm.at[sidx_v.at[j]],
        rows_v.at[pl.ds(j * CHUNK, CHUNK)], sem) for j in range(N_CHUNK)]
    for cp in copies:
        cp.wait()
    pltpu.sync_copy(rows_v, srow_out.at[pl.ds(base, B_PER_W)])
    rows_i = rows_v.bitcast(jnp.int32)
    copies = [pltpu.async_copy(
        gq_hbm.at[qidx_v.at[j]],
        rows_i.at[pl.ds(j * CHUNK, CHUNK)], sem) for j in range(N_CHUNK)]
    for cp in copies:
        cp.wait()
    pltpu.sync_copy(rows_i, dpack_out.at[pl.ds(base, B_PER_W)])


def _sc_gather(gq_table, qidx2d, t_table, sidx2d):
    mesh = plsc.VectorSubcoreMesh(core_axis_name="c", subcore_axis_name="s")
    return pl.kernel(
        _sc_gather_body,
        mesh=mesh,
        out_type=[
            jax.ShapeDtypeStruct((B, T_ROW), jnp.int32),
            jax.ShapeDtypeStruct((B, T_ROW), jnp.float32),
        ],
        scratch_types=[
            pltpu.VMEM((N_CHUNK, CHUNK), jnp.int32),
            pltpu.VMEM((N_CHUNK, CHUNK), jnp.int32),
            pltpu.VMEM((B_PER_W, T_ROW), jnp.float32),
            pltpu.SemaphoreType.DMA,
        ],
    )(gq_table, qidx2d, t_table, sidx2d)


_MLP_BLOCK = 2048


def _mlp_body(sg_ref, gq_ref, sel_ref, bd1_ref, wd2_ref, bd2_ref,
              sout_ref, dout_ref):
    sout_ref[...] = sg_ref[:, :OUT]
    g = lax.bitcast_convert_type(gq_ref[...], jnp.uint32)
    sel = sel_ref[...]
    half = jnp.where(sel >= 2, g[:, HID_PAD:], g[:, :HID_PAD])
    bits = jnp.where((sel & 1) == 1, half << 16, (half >> 16) << 16)
    x = lax.bitcast_convert_type(bits, jnp.float32)
    h = jnp.maximum(x[:, :HID] + bd1_ref[...], 0.0)
    dout_ref[...] = (
        jnp.dot(h, wd2_ref[...], preferred_element_type=jnp.float32)
        + bd2_ref[...]
    )


def _mlp(s_rows, d_pack, sel, bd1, wd2, bd2):
    nblk = B // _MLP_BLOCK
    return pl.pallas_call(
        _mlp_body,
        grid=(nblk,),
        in_specs=[
            pl.BlockSpec((_MLP_BLOCK, T_ROW), lambda i: (i, 0)),
            pl.BlockSpec((_MLP_BLOCK, T_ROW), lambda i: (i, 0)),
            pl.BlockSpec((_MLP_BLOCK, 1), lambda i: (i, 0)),
            pl.BlockSpec((1, HID), lambda i: (0, 0)),
            pl.BlockSpec((HID, OUT), lambda i: (0, 0)),
            pl.BlockSpec((1, OUT), lambda i: (0, 0)),
        ],
        out_specs=[
            pl.BlockSpec((_MLP_BLOCK, OUT), lambda i: (i, 0)),
            pl.BlockSpec((_MLP_BLOCK, OUT), lambda i: (i, 0)),
        ],
        out_shape=[
            jax.ShapeDtypeStruct((B, OUT), jnp.float32),
            jax.ShapeDtypeStruct((B, OUT), jnp.float32),
        ],
    )(s_rows, d_pack, sel, bd1, wd2, bd2)


def kernel(sparse_col_inp, dense_col_inp, emb_table, Ws1, bs1, Ws2, bs2,
           Wd1, bd1, Wd2, bd2):
    sidx = sparse_col_inp.astype(jnp.int32).reshape(B // CHUNK, CHUNK)
    didx = dense_col_inp.astype(jnp.int32)
    qidx = (didx >> 2).reshape(B // CHUNK, CHUNK)
    sel = (didx & 3).reshape(B, 1)
    ws1p = jnp.pad(Ws1, ((0, VOCAB_PAD - SPARSE_VOCAB), (0, 0)))
    ws2p = jnp.pad(Ws2, ((0, 0), (0, T_ROW - OUT)))
    bs2p = jnp.pad(bs2, (0, T_ROW - OUT))
    wd1p = jnp.pad(Wd1, ((0, 0), (0, HID_PAD - HID)))
    t_table = _precompute_table(ws1p, bs1.reshape(1, HID),
                                ws2p, bs2p.reshape(1, T_ROW))
    gq_table = _gbuild(emb_table.T, wd1p)
    d_pack, s_rows = _sc_gather(gq_table, qidx, t_table, sidx)
    sparse_out, dense_out = _mlp(s_rows, d_pack, sel,
                                 bd1.reshape(1, HID), Wd2,
                                 bd2.reshape(1, OUT))
    return (sparse_out, dense_out)


# speedup vs baseline: 2.2194x; 1.4484x over previous
import jax
import jax.numpy as jnp
from jax import lax
from jax.experimental import pallas as pl
from jax.experimental.pallas import tpu as pltpu
from jax.experimental.pallas import tpu_sc as plsc

B = 16384
SPARSE_VOCAB = 1000
VOCAB_PAD = 1024
DENSE_VOCAB = 1000000
EMB_DIM = 64
HID = 50
OUT = 50
T_ROW = 128
HID_PAD = 64
QUAD = 4
GQ_ROWS = 262144
Q_SHIFT = 18

NC = 2
NS = 16
NW = NC * NS
B_PER_W = B // NW
CHUNK = 128
N_CHUNK = B_PER_W // CHUNK


def _precompute_body(ws1_ref, bs1_ref, ws2_ref, bs2_ref, t_ref):
    h = jnp.maximum(ws1_ref[...] + bs1_ref[...], 0.0)
    t_ref[...] = (
        jnp.dot(h, ws2_ref[...], preferred_element_type=jnp.float32)
        + bs2_ref[...]
    )


def _precompute_table(ws1p, bs1, ws2p, bs2p):
    return pl.pallas_call(
        _precompute_body,
        out_shape=jax.ShapeDtypeStruct((VOCAB_PAD, T_ROW), jnp.float32),
    )(ws1p, bs1, ws2p, bs2p)


_GBLK = 2048
_GOFF = GQ_ROWS // _GBLK
_GMAX = (DENSE_VOCAB - 1) // _GBLK


def _gbuild_body(et0_ref, et1_ref, et2_ref, et3_ref, wd1_ref, gq_ref):
    def dot_t(ref):
        x = lax.dot_general(ref[...], wd1_ref[...], (((0,), (0,)), ((), ())),
                            preferred_element_type=jnp.float32)
        return lax.bitcast_convert_type(x, jnp.uint32)

    b0, b1, b2, b3 = dot_t(et0_ref), dot_t(et1_ref), dot_t(et2_ref), \
        dot_t(et3_ref)
    p01 = ((b0 >> 16) << 16) | (b1 >> 16)
    p23 = ((b2 >> 16) << 16) | (b3 >> 16)
    gq_ref[...] = lax.bitcast_convert_type(
        jnp.concatenate([p01, p23], axis=1), jnp.int32)


def _gbuild(emb_t, wd1p):
    nblk = GQ_ROWS // _GBLK
    return pl.pallas_call(
        _gbuild_body,
        grid=(nblk,),
        in_specs=[
            pl.BlockSpec((EMB_DIM, _GBLK), lambda i: (0, i)),
            pl.BlockSpec((EMB_DIM, _GBLK), lambda i: (0, i + _GOFF)),
            pl.BlockSpec((EMB_DIM, _GBLK), lambda i: (0, i + 2 * _GOFF)),
            pl.BlockSpec((EMB_DIM, _GBLK),
                         lambda i: (0, jnp.minimum(i + 3 * _GOFF, _GMAX))),
            pl.BlockSpec((EMB_DIM, HID_PAD), lambda i: (0, 0)),
        ],
        out_specs=pl.BlockSpec((_GBLK, 2 * HID_PAD), lambda i: (i, 0)),
        out_shape=jax.ShapeDtypeStruct((GQ_ROWS, 2 * HID_PAD), jnp.int32),
    )(emb_t, emb_t, emb_t, emb_t, wd1p)


def _sc_gather_body(gq_hbm, qidx_hbm, t_hbm, sidx_hbm,
                    dpack_out, srow_out,
                    qidx_v, sidx_v, rows_v, sem):
    wid = lax.axis_index("s") * NC + lax.axis_index("c")
    base = wid * B_PER_W
    row0 = wid * N_CHUNK
    pltpu.sync_copy(qidx_hbm.at[pl.ds(row0, N_CHUNK)], qidx_v)
    pltpu.sync_copy(sidx_hbm.at[pl.ds(row0, N_CHUNK)], sidx_v)
    copies = [pltpu.async_copy(
        t_hbm.at[sidx_v.at[j]],
        rows_v.at[pl.ds(j * CHUNK, CHUNK)], sem) for j in range(N_CHUNK)]
    for cp in copies:
        cp.wait()
    pltpu.sync_copy(rows_v, srow_out.at[pl.ds(base, B_PER_W)])
    rows_i = rows_v.bitcast(jnp.int32)
    copies = [pltpu.async_copy(
        gq_hbm.at[qidx_v.at[j]],
        rows_i.at[pl.ds(j * CHUNK, CHUNK)], sem) for j in range(N_CHUNK)]
    for cp in copies:
        cp.wait()
    pltpu.sync_copy(rows_i, dpack_out.at[pl.ds(base, B_PER_W)])


def _sc_gather(gq_table, qidx2d, t_table, sidx2d):
    mesh = plsc.VectorSubcoreMesh(core_axis_name="c", subcore_axis_name="s")
    return pl.kernel(
        _sc_gather_body,
        mesh=mesh,
        out_type=[
            jax.ShapeDtypeStruct((B, T_ROW), jnp.int32),
            jax.ShapeDtypeStruct((B, T_ROW), jnp.float32),
        ],
        scratch_types=[
            pltpu.VMEM((N_CHUNK, CHUNK), jnp.int32),
            pltpu.VMEM((N_CHUNK, CHUNK), jnp.int32),
            pltpu.VMEM((B_PER_W, T_ROW), jnp.float32),
            pltpu.SemaphoreType.DMA,
        ],
    )(gq_table, qidx2d, t_table, sidx2d)


_MLP_BLOCK = 2048


def _mlp_body(sg_ref, gq_ref, sel_ref, bd1_ref, wd2_ref, bd2_ref,
              sout_ref, dout_ref):
    sout_ref[...] = sg_ref[:, :OUT]
    g = lax.bitcast_convert_type(gq_ref[...], jnp.uint32)
    sel = sel_ref[...]
    half = jnp.where(sel >= 2, g[:, HID_PAD:], g[:, :HID_PAD])
    bits = jnp.where((sel & 1) == 1, half << 16, (half >> 16) << 16)
    x = lax.bitcast_convert_type(bits, jnp.float32)
    h = jnp.maximum(x[:, :HID] + bd1_ref[...], 0.0)
    dout_ref[...] = (
        jnp.dot(h, wd2_ref[...], preferred_element_type=jnp.float32)
        + bd2_ref[...]
    )


def _mlp(s_rows, d_pack, sel, bd1, wd2, bd2):
    nblk = B // _MLP_BLOCK
    return pl.pallas_call(
        _mlp_body,
        grid=(nblk,),
        in_specs=[
            pl.BlockSpec((_MLP_BLOCK, T_ROW), lambda i: (i, 0)),
            pl.BlockSpec((_MLP_BLOCK, T_ROW), lambda i: (i, 0)),
            pl.BlockSpec((_MLP_BLOCK, 1), lambda i: (i, 0)),
            pl.BlockSpec((1, HID), lambda i: (0, 0)),
            pl.BlockSpec((HID, OUT), lambda i: (0, 0)),
            pl.BlockSpec((1, OUT), lambda i: (0, 0)),
        ],
        out_specs=[
            pl.BlockSpec((_MLP_BLOCK, OUT), lambda i: (i, 0)),
            pl.BlockSpec((_MLP_BLOCK, OUT), lambda i: (i, 0)),
        ],
        out_shape=[
            jax.ShapeDtypeStruct((B, OUT), jnp.float32),
            jax.ShapeDtypeStruct((B, OUT), jnp.float32),
        ],
    )(s_rows, d_pack, sel, bd1, wd2, bd2)


def kernel(sparse_col_inp, dense_col_inp, emb_table, Ws1, bs1, Ws2, bs2,
           Wd1, bd1, Wd2, bd2):
    sidx = sparse_col_inp.astype(jnp.int32).reshape(B // CHUNK, CHUNK)
    didx = dense_col_inp.astype(jnp.int32)
    qidx = (didx & (GQ_ROWS - 1)).reshape(B // CHUNK, CHUNK)
    sel = (didx >> Q_SHIFT).reshape(B, 1)
    ws1p = jnp.pad(Ws1, ((0, VOCAB_PAD - SPARSE_VOCAB), (0, 0)))
    ws2p = jnp.pad(Ws2, ((0, 0), (0, T_ROW - OUT)))
    bs2p = jnp.pad(bs2, (0, T_ROW - OUT))
    wd1p = jnp.pad(Wd1, ((0, 0), (0, HID_PAD - HID)))
    t_table = _precompute_table(ws1p, bs1.reshape(1, HID),
                                ws2p, bs2p.reshape(1, T_ROW))
    gq_table = _gbuild(emb_table.T, wd1p)
    d_pack, s_rows = _sc_gather(gq_table, qidx, t_table, sidx)
    sparse_out, dense_out = _mlp(s_rows, d_pack, sel,
                                 bd1.reshape(1, HID), Wd2,
                                 bd2.reshape(1, OUT))
    return (sparse_out, dense_out)


# speedup vs baseline: 2.5221x; 1.1364x over previous
import jax
import jax.numpy as jnp
from jax import lax
from jax.experimental import pallas as pl
from jax.experimental.pallas import tpu as pltpu
from jax.experimental.pallas import tpu_sc as plsc

B = 16384
SPARSE_VOCAB = 1000
VOCAB_PAD = 1024
DENSE_VOCAB = 1000000
EMB_DIM = 64
HID = 50
OUT = 50
T_ROW = 128
HID_PAD = 64
QUAD = 4
GQ_ROWS = 262144
Q_SHIFT = 18

NC = 2
NS = 16
NW = NC * NS
B_PER_W = B // NW
CHUNK = 128
N_CHUNK = B_PER_W // CHUNK


def _precompute_body(ws1_ref, bs1_ref, ws2_ref, bs2_ref, t_ref):
    h = jnp.maximum(ws1_ref[...] + bs1_ref[...], 0.0)
    t_ref[...] = (
        jnp.dot(h, ws2_ref[...], preferred_element_type=jnp.float32)
        + bs2_ref[...]
    )


def _precompute_table(ws1p, bs1, ws2p, bs2p):
    return pl.pallas_call(
        _precompute_body,
        out_shape=jax.ShapeDtypeStruct((VOCAB_PAD, T_ROW), jnp.float32),
    )(ws1p, bs1, ws2p, bs2p)


_GBLK = 4096
_GOFF = GQ_ROWS // _GBLK
_GMAX = (DENSE_VOCAB - 1) // _GBLK


def _gbuild_body(et0_ref, et1_ref, et2_ref, et3_ref, wd1_ref, gq_ref):
    def dot_t(ref):
        x = lax.dot_general(ref[...], wd1_ref[...], (((0,), (0,)), ((), ())),
                            preferred_element_type=jnp.float32)
        return lax.bitcast_convert_type(x, jnp.uint32)

    b0, b1, b2, b3 = dot_t(et0_ref), dot_t(et1_ref), dot_t(et2_ref), \
        dot_t(et3_ref)
    p01 = ((b0 >> 16) << 16) | (b1 >> 16)
    p23 = ((b2 >> 16) << 16) | (b3 >> 16)
    gq_ref[...] = lax.bitcast_convert_type(
        jnp.concatenate([p01, p23], axis=1), jnp.int32)


def _gbuild(emb_t, wd1p):
    nblk = GQ_ROWS // _GBLK
    return pl.pallas_call(
        _gbuild_body,
        grid=(nblk,),
        in_specs=[
            pl.BlockSpec((EMB_DIM, _GBLK), lambda i: (0, i)),
            pl.BlockSpec((EMB_DIM, _GBLK), lambda i: (0, i + _GOFF)),
            pl.BlockSpec((EMB_DIM, _GBLK), lambda i: (0, i + 2 * _GOFF)),
            pl.BlockSpec((EMB_DIM, _GBLK),
                         lambda i: (0, jnp.minimum(i + 3 * _GOFF, _GMAX))),
            pl.BlockSpec((EMB_DIM, HID_PAD), lambda i: (0, 0)),
        ],
        out_specs=pl.BlockSpec((_GBLK, 2 * HID_PAD), lambda i: (i, 0)),
        out_shape=jax.ShapeDtypeStruct((GQ_ROWS, 2 * HID_PAD), jnp.int32),
    )(emb_t, emb_t, emb_t, emb_t, wd1p)


def _sc_gather_body(gq_hbm, qidx_hbm, t_hbm, sidx_hbm,
                    dpack_out, srow_out,
                    qidx_v, sidx_v, rows_v, sem):
    wid = lax.axis_index("s") * NC + lax.axis_index("c")
    base = wid * B_PER_W
    row0 = wid * N_CHUNK
    pltpu.sync_copy(qidx_hbm.at[pl.ds(row0, N_CHUNK)], qidx_v)
    pltpu.sync_copy(sidx_hbm.at[pl.ds(row0, N_CHUNK)], sidx_v)
    copies = [pltpu.async_copy(
        t_hbm.at[sidx_v.at[j]],
        rows_v.at[pl.ds(j * CHUNK, CHUNK)], sem) for j in range(N_CHUNK)]
    for cp in copies:
        cp.wait()
    pltpu.sync_copy(rows_v, srow_out.at[pl.ds(base, B_PER_W)])
    rows_i = rows_v.bitcast(jnp.int32)
    copies = [pltpu.async_copy(
        gq_hbm.at[qidx_v.at[j]],
        rows_i.at[pl.ds(j * CHUNK, CHUNK)], sem) for j in range(N_CHUNK)]
    for cp in copies:
        cp.wait()
    pltpu.sync_copy(rows_i, dpack_out.at[pl.ds(base, B_PER_W)])


def _sc_gather(gq_table, qidx2d, t_table, sidx2d):
    mesh = plsc.VectorSubcoreMesh(core_axis_name="c", subcore_axis_name="s")
    return pl.kernel(
        _sc_gather_body,
        mesh=mesh,
        out_type=[
            jax.ShapeDtypeStruct((B, T_ROW), jnp.int32),
            jax.ShapeDtypeStruct((B, T_ROW), jnp.float32),
        ],
        scratch_types=[
            pltpu.VMEM((N_CHUNK, CHUNK), jnp.int32),
            pltpu.VMEM((N_CHUNK, CHUNK), jnp.int32),
            pltpu.VMEM((B_PER_W, T_ROW), jnp.float32),
            pltpu.SemaphoreType.DMA,
        ],
    )(gq_table, qidx2d, t_table, sidx2d)


_MLP_BLOCK = 2048


def _mlp_body(sg_ref, gq_ref, sel_ref, bd1_ref, wd2_ref, bd2_ref,
              sout_ref, dout_ref):
    sout_ref[...] = sg_ref[:, :OUT]
    g = lax.bitcast_convert_type(gq_ref[...], jnp.uint32)
    sel = sel_ref[...]
    half = jnp.where(sel >= 2, g[:, HID_PAD:], g[:, :HID_PAD])
    bits = jnp.where((sel & 1) == 1, half << 16, (half >> 16) << 16)
    x = lax.bitcast_convert_type(bits, jnp.float32)
    h = jnp.maximum(x[:, :HID] + bd1_ref[...], 0.0)
    dout_ref[...] = (
        jnp.dot(h, wd2_ref[...], preferred_element_type=jnp.float32)
        + bd2_ref[...]
    )


def _mlp(s_rows, d_pack, sel, bd1, wd2, bd2):
    nblk = B // _MLP_BLOCK
    return pl.pallas_call(
        _mlp_body,
        grid=(nblk,),
        in_specs=[
            pl.BlockSpec((_MLP_BLOCK, T_ROW), lambda i: (i, 0)),
            pl.BlockSpec((_MLP_BLOCK, T_ROW), lambda i: (i, 0)),
            pl.BlockSpec((_MLP_BLOCK, 1), lambda i: (i, 0)),
            pl.BlockSpec((1, HID), lambda i: (0, 0)),
            pl.BlockSpec((HID, OUT), lambda i: (0, 0)),
            pl.BlockSpec((1, OUT), lambda i: (0, 0)),
        ],
        out_specs=[
            pl.BlockSpec((_MLP_BLOCK, OUT), lambda i: (i, 0)),
            pl.BlockSpec((_MLP_BLOCK, OUT), lambda i: (i, 0)),
        ],
        out_shape=[
            jax.ShapeDtypeStruct((B, OUT), jnp.float32),
            jax.ShapeDtypeStruct((B, OUT), jnp.float32),
        ],
    )(s_rows, d_pack, sel, bd1, wd2, bd2)


def kernel(sparse_col_inp, dense_col_inp, emb_table, Ws1, bs1, Ws2, bs2,
           Wd1, bd1, Wd2, bd2):
    sidx = sparse_col_inp.astype(jnp.int32).reshape(B // CHUNK, CHUNK)
    didx = dense_col_inp.astype(jnp.int32)
    qidx = (didx & (GQ_ROWS - 1)).reshape(B // CHUNK, CHUNK)
    sel = (didx >> Q_SHIFT).reshape(B, 1)
    ws1p = jnp.pad(Ws1, ((0, VOCAB_PAD - SPARSE_VOCAB), (0, 0)))
    ws2p = jnp.pad(Ws2, ((0, 0), (0, T_ROW - OUT)))
    bs2p = jnp.pad(bs2, (0, T_ROW - OUT))
    wd1p = jnp.pad(Wd1, ((0, 0), (0, HID_PAD - HID)))
    t_table = _precompute_table(ws1p, bs1.reshape(1, HID),
                                ws2p, bs2p.reshape(1, T_ROW))
    gq_table = _gbuild(emb_table.T, wd1p)
    d_pack, s_rows = _sc_gather(gq_table, qidx, t_table, sidx)
    sparse_out, dense_out = _mlp(s_rows, d_pack, sel,
                                 bd1.reshape(1, HID), Wd2,
                                 bd2.reshape(1, OUT))
    return (sparse_out, dense_out)


# speedup vs baseline: 2.6002x; 1.0310x over previous
import jax
import jax.numpy as jnp
from jax import lax
from jax.experimental import pallas as pl
from jax.experimental.pallas import tpu as pltpu
from jax.experimental.pallas import tpu_sc as plsc

B = 16384
SPARSE_VOCAB = 1000
VOCAB_PAD = 1024
DENSE_VOCAB = 1000000
EMB_DIM = 64
HID = 50
OUT = 50
T_ROW = 128
HID_PAD = 64
QUAD = 4
GQ_ROWS = 262144
Q_SHIFT = 18

NC = 2
NS = 16
NW = NC * NS
B_PER_W = B // NW
CHUNK = 128
N_CHUNK = B_PER_W // CHUNK


def _precompute_body(ws1_ref, bs1_ref, ws2_ref, bs2_ref, t_ref):
    h = jnp.maximum(ws1_ref[...] + bs1_ref[...], 0.0)
    t_ref[...] = (
        jnp.dot(h, ws2_ref[...], preferred_element_type=jnp.float32)
        + bs2_ref[...]
    )


def _precompute_table(ws1p, bs1, ws2p, bs2p):
    return pl.pallas_call(
        _precompute_body,
        out_shape=jax.ShapeDtypeStruct((VOCAB_PAD, T_ROW), jnp.float32),
    )(ws1p, bs1, ws2p, bs2p)


_GBLK = 8192
_GOFF = GQ_ROWS // _GBLK
_GMAX = (DENSE_VOCAB - 1) // _GBLK


def _gbuild_body(et0_ref, et1_ref, et2_ref, et3_ref, wd1_ref, gq_ref):
    def dot_t(ref):
        x = lax.dot_general(ref[...], wd1_ref[...], (((0,), (0,)), ((), ())),
                            preferred_element_type=jnp.float32)
        return lax.bitcast_convert_type(x, jnp.uint32)

    b0, b1, b2, b3 = dot_t(et0_ref), dot_t(et1_ref), dot_t(et2_ref), \
        dot_t(et3_ref)
    p01 = ((b0 >> 16) << 16) | (b1 >> 16)
    p23 = ((b2 >> 16) << 16) | (b3 >> 16)
    gq_ref[...] = lax.bitcast_convert_type(
        jnp.concatenate([p01, p23], axis=1), jnp.int32)


def _gbuild(emb_t, wd1p):
    nblk = GQ_ROWS // _GBLK
    return pl.pallas_call(
        _gbuild_body,
        grid=(nblk,),
        in_specs=[
            pl.BlockSpec((EMB_DIM, _GBLK), lambda i: (0, i)),
            pl.BlockSpec((EMB_DIM, _GBLK), lambda i: (0, i + _GOFF)),
            pl.BlockSpec((EMB_DIM, _GBLK), lambda i: (0, i + 2 * _GOFF)),
            pl.BlockSpec((EMB_DIM, _GBLK),
                         lambda i: (0, jnp.minimum(i + 3 * _GOFF, _GMAX))),
            pl.BlockSpec((EMB_DIM, HID_PAD), lambda i: (0, 0)),
        ],
        out_specs=pl.BlockSpec((_GBLK, 2 * HID_PAD), lambda i: (i, 0)),
        out_shape=jax.ShapeDtypeStruct((GQ_ROWS, 2 * HID_PAD), jnp.int32),
    )(emb_t, emb_t, emb_t, emb_t, wd1p)


def _sc_gather_body(gq_hbm, qidx_hbm, t_hbm, sidx_hbm,
                    dpack_out, srow_out,
                    qidx_v, sidx_v, rows_v, sem):
    wid = lax.axis_index("s") * NC + lax.axis_index("c")
    base = wid * B_PER_W
    row0 = wid * N_CHUNK
    pltpu.sync_copy(qidx_hbm.at[pl.ds(row0, N_CHUNK)], qidx_v)
    pltpu.sync_copy(sidx_hbm.at[pl.ds(row0, N_CHUNK)], sidx_v)
    copies = [pltpu.async_copy(
        t_hbm.at[sidx_v.at[j]],
        rows_v.at[pl.ds(j * CHUNK, CHUNK)], sem) for j in range(N_CHUNK)]
    for cp in copies:
        cp.wait()
    pltpu.sync_copy(rows_v, srow_out.at[pl.ds(base, B_PER_W)])
    rows_i = rows_v.bitcast(jnp.int32)
    copies = [pltpu.async_copy(
        gq_hbm.at[qidx_v.at[j]],
        rows_i.at[pl.ds(j * CHUNK, CHUNK)], sem) for j in range(N_CHUNK)]
    for cp in copies:
        cp.wait()
    pltpu.sync_copy(rows_i, dpack_out.at[pl.ds(base, B_PER_W)])


def _sc_gather(gq_table, qidx2d, t_table, sidx2d):
    mesh = plsc.VectorSubcoreMesh(core_axis_name="c", subcore_axis_name="s")
    return pl.kernel(
        _sc_gather_body,
        mesh=mesh,
        out_type=[
            jax.ShapeDtypeStruct((B, T_ROW), jnp.int32),
            jax.ShapeDtypeStruct((B, T_ROW), jnp.float32),
        ],
        scratch_types=[
            pltpu.VMEM((N_CHUNK, CHUNK), jnp.int32),
            pltpu.VMEM((N_CHUNK, CHUNK), jnp.int32),
            pltpu.VMEM((B_PER_W, T_ROW), jnp.float32),
            pltpu.SemaphoreType.DMA,
        ],
    )(gq_table, qidx2d, t_table, sidx2d)


_MLP_BLOCK = 2048


def _mlp_body(sg_ref, gq_ref, sel_ref, bd1_ref, wd2_ref, bd2_ref,
              sout_ref, dout_ref):
    sout_ref[...] = sg_ref[:, :OUT]
    g = lax.bitcast_convert_type(gq_ref[...], jnp.uint32)
    sel = sel_ref[...]
    half = jnp.where(sel >= 2, g[:, HID_PAD:], g[:, :HID_PAD])
    bits = jnp.where((sel & 1) == 1, half << 16, (half >> 16) << 16)
    x = lax.bitcast_convert_type(bits, jnp.float32)
    h = jnp.maximum(x[:, :HID] + bd1_ref[...], 0.0)
    dout_ref[...] = (
        jnp.dot(h, wd2_ref[...], preferred_element_type=jnp.float32)
        + bd2_ref[...]
    )


def _mlp(s_rows, d_pack, sel, bd1, wd2, bd2):
    nblk = B // _MLP_BLOCK
    return pl.pallas_call(
        _mlp_body,
        grid=(nblk,),
        in_specs=[
            pl.BlockSpec((_MLP_BLOCK, T_ROW), lambda i: (i, 0)),
            pl.BlockSpec((_MLP_BLOCK, T_ROW), lambda i: (i, 0)),
            pl.BlockSpec((_MLP_BLOCK, 1), lambda i: (i, 0)),
            pl.BlockSpec((1, HID), lambda i: (0, 0)),
            pl.BlockSpec((HID, OUT), lambda i: (0, 0)),
            pl.BlockSpec((1, OUT), lambda i: (0, 0)),
        ],
        out_specs=[
            pl.BlockSpec((_MLP_BLOCK, OUT), lambda i: (i, 0)),
            pl.BlockSpec((_MLP_BLOCK, OUT), lambda i: (i, 0)),
        ],
        out_shape=[
            jax.ShapeDtypeStruct((B, OUT), jnp.float32),
            jax.ShapeDtypeStruct((B, OUT), jnp.float32),
        ],
    )(s_rows, d_pack, sel, bd1, wd2, bd2)


def kernel(sparse_col_inp, dense_col_inp, emb_table, Ws1, bs1, Ws2, bs2,
           Wd1, bd1, Wd2, bd2):
    sidx = sparse_col_inp.astype(jnp.int32).reshape(B // CHUNK, CHUNK)
    didx = dense_col_inp.astype(jnp.int32)
    qidx = (didx & (GQ_ROWS - 1)).reshape(B // CHUNK, CHUNK)
    sel = (didx >> Q_SHIFT).reshape(B, 1)
    ws1p = jnp.pad(Ws1, ((0, VOCAB_PAD - SPARSE_VOCAB), (0, 0)))
    ws2p = jnp.pad(Ws2, ((0, 0), (0, T_ROW - OUT)))
    bs2p = jnp.pad(bs2, (0, T_ROW - OUT))
    wd1p = jnp.pad(Wd1, ((0, 0), (0, HID_PAD - HID)))
    t_table = _precompute_table(ws1p, bs1.reshape(1, HID),
                                ws2p, bs2p.reshape(1, T_ROW))
    gq_table = _gbuild(emb_table.T, wd1p)
    d_pack, s_rows = _sc_gather(gq_table, qidx, t_table, sidx)
    sparse_out, dense_out = _mlp(s_rows, d_pack, sel,
                                 bd1.reshape(1, HID), Wd2,
                                 bd2.reshape(1, OUT))
    return (sparse_out, dense_out)
